# initial kernel scaffold (unmeasured)
import jax
import jax.numpy as jnp
from jax import lax
from jax.experimental import pallas as pl
from jax.experimental.pallas import tpu as pltpu

N_DEV = 32


def kernel(x, w_mat):
    m, k = x.shape
    _, n = w_mat.shape
    ch = m // N_DEV

    def body(x_ref, w_ref, out_ref, acc_ref, comm_ref,
             send_sems, recv_sems, credit_sem):
        my = lax.axis_index("i")
        left = lax.rem(my + N_DEV - 1, N_DEV)
        right = lax.rem(my + 1, N_DEV)

        barrier = pltpu.get_barrier_semaphore()
        for nbr in (left, right):
            pl.semaphore_signal(barrier, inc=1, device_id=(nbr,),
                                device_id_type=pl.DeviceIdType.MESH)
        pl.semaphore_wait(barrier, 2)

        acc_ref[...] = lax.dot_general(
            x_ref[...].astype(jnp.bfloat16),
            w_ref[...].astype(jnp.bfloat16),
            dimension_numbers=(((1,), (0,)), ((), ())),
            preferred_element_type=jnp.float32,
        )

        def ring_step(s, send_chunk, recv_chunk, src_ref, use_credit):
            slot = s % 2
            if use_credit:
                pl.semaphore_wait(credit_sem, 1)
            rdma = pltpu.make_async_remote_copy(
                src_ref=src_ref.at[pl.ds(send_chunk * ch, ch), :],
                dst_ref=comm_ref.at[slot],
                send_sem=send_sems.at[slot],
                recv_sem=recv_sems.at[slot],
                device_id=(right,),
                device_id_type=pl.DeviceIdType.MESH,
            )
            rdma.start()
            rdma.wait()
            return slot

        for s in range(N_DEV - 1):
            send_chunk = lax.rem(my - s + 2 * N_DEV, N_DEV)
            recv_chunk = lax.rem(my - s - 1 + 2 * N_DEV, N_DEV)
            slot = ring_step(s, send_chunk, recv_chunk, acc_ref, s >= 2)
            acc_ref[pl.ds(recv_chunk * ch, ch), :] += comm_ref[slot]
            pl.semaphore_signal(credit_sem, inc=1, device_id=(left,),
                                device_id_type=pl.DeviceIdType.MESH)

        own = lax.rem(my + 1, N_DEV)
        y = acc_ref[pl.ds(own * ch, ch), :]
        out_ref[pl.ds(own * ch, ch), :] = y * jax.nn.sigmoid(y)

        for t in range(N_DEV - 1):
            s = N_DEV - 1 + t
            send_chunk = lax.rem(my + 1 - t + 2 * N_DEV, N_DEV)
            recv_chunk = lax.rem(my - t + 2 * N_DEV, N_DEV)
            slot = ring_step(s, send_chunk, recv_chunk, out_ref, True)
            out_ref[pl.ds(recv_chunk * ch, ch), :] = comm_ref[slot]
            pl.semaphore_signal(credit_sem, inc=1, device_id=(left,),
                                device_id_type=pl.DeviceIdType.MESH)

        pl.semaphore_wait(credit_sem, 2)

    return pl.pallas_call(
        body,
        out_shape=jax.ShapeDtypeStruct((m, n), jnp.float32),
        in_specs=[
            pl.BlockSpec(memory_space=pltpu.VMEM),
            pl.BlockSpec(memory_space=pltpu.VMEM),
        ],
        out_specs=pl.BlockSpec(memory_space=pltpu.VMEM),
        scratch_shapes=[
            pltpu.VMEM((m, n), jnp.float32),
            pltpu.VMEM((2, ch, n), jnp.float32),
            pltpu.SemaphoreType.DMA((2,)),
            pltpu.SemaphoreType.DMA((2,)),
            pltpu.SemaphoreType.REGULAR,
        ],
        compiler_params=pltpu.CompilerParams(collective_id=0),
    )(x, w_mat)


# baseline (device time: 695723 ns/iter reference)
import numpy as np

import jax
import jax.numpy as jnp
from jax import lax
from jax.experimental import pallas as pl
from jax.experimental.pallas import tpu as pltpu

N_DEV = 32
N_STEPS = 7

_COORDS = [
    (0, 0, 0), (1, 0, 0), (1, 1, 0), (0, 1, 0),
    (0, 2, 0), (1, 2, 0), (1, 3, 0), (0, 3, 0),
    (0, 0, 1), (1, 0, 1), (1, 1, 1), (0, 1, 1),
    (0, 2, 1), (1, 2, 1), (1, 3, 1), (0, 3, 1),
    (0, 0, 2), (1, 0, 2), (1, 1, 2), (0, 1, 2),
    (0, 2, 2), (1, 2, 2), (1, 3, 2), (0, 3, 2),
    (0, 0, 3), (1, 0, 3), (1, 1, 3), (0, 1, 3),
    (0, 2, 3), (1, 2, 3), (1, 3, 3), (0, 3, 3),
]
_BY_C = {c: l for l, c in enumerate(_COORDS)}

ADD, REPLACE, DISCARD = 0, 1, 2


def _schedule():
    ptn = np.zeros((N_DEV, N_STEPS), np.int32)
    opc = np.zeros((N_DEV, N_STEPS), np.int32)
    for l, (x, y, z) in enumerate(_COORDS):
        xp = _BY_C[(1 - x, y, z)]
        ptn[l, 0], opc[l, 0] = xp, ADD
        ptn[l, 1], opc[l, 1] = _BY_C[(x, y ^ 1, z)], ADD
        if y in (1, 2):
            ptn[l, 2], opc[l, 2] = _BY_C[(x, 3 - y, z)], ADD
        else:
            ptn[l, 2], opc[l, 2] = xp, DISCARD
        ptn[l, 3] = _BY_C[(x, y ^ 1, z)]
        opc[l, 3] = REPLACE if y in (0, 3) else DISCARD
        ptn[l, 4], opc[l, 4] = _BY_C[(x, y, z ^ 1)], ADD
        if z in (1, 2):
            ptn[l, 5], opc[l, 5] = _BY_C[(x, y, 3 - z)], ADD
        else:
            ptn[l, 5], opc[l, 5] = xp, DISCARD
        ptn[l, 6] = _BY_C[(x, y, z ^ 1)]
        opc[l, 6] = REPLACE if z in (0, 3) else DISCARD
    return ptn, opc


_PTN, _OPC = _schedule()


def kernel(x, w_mat):
    m, k = x.shape
    _, n = w_mat.shape

    idx = lax.axis_index("i")
    ptn = jnp.take(jnp.asarray(_PTN), idx, axis=0)
    opc = jnp.take(jnp.asarray(_OPC), idx, axis=0)

    def body(ptn_ref, opc_ref, x_ref, w_ref, out_ref,
             sbuf, bufs, tsrc, tdst,
             send_sems, recv_sems, tsend, trecv):
        barrier = pltpu.get_barrier_semaphore()
        for s in range(N_STEPS):
            pl.semaphore_signal(barrier, inc=1, device_id=(ptn_ref[s],),
                                device_id_type=pl.DeviceIdType.MESH)
        pl.semaphore_wait(barrier, N_STEPS)

        out_ref[...] = lax.dot_general(
            x_ref[...].astype(jnp.bfloat16),
            w_ref[...].astype(jnp.bfloat16),
            dimension_numbers=(((1,), (0,)), ((), ())),
            preferred_element_type=jnp.float32,
        )

        for s in range(1, N_STEPS + 1):
            si = s - 1
            slot = s % 2
            if s >= 3:
                crw = pltpu.make_async_remote_copy(
                    src_ref=tsrc, dst_ref=tdst.at[si],
                    send_sem=tsend.at[si], recv_sem=trecv.at[si],
                    device_id=(ptn_ref[si],),
                    device_id_type=pl.DeviceIdType.MESH,
                )
                crw.wait_recv()
            sbuf[...] = out_ref[...].astype(jnp.bfloat16)
            rdma = pltpu.make_async_remote_copy(
                src_ref=sbuf, dst_ref=bufs.at[slot],
                send_sem=send_sems.at[si], recv_sem=recv_sems.at[si],
                device_id=(ptn_ref[si],),
                device_id_type=pl.DeviceIdType.MESH,
            )
            rdma.start()
            rdma.wait()

            op = opc_ref[si]

            @pl.when(op == ADD)
            def _():
                out_ref[...] += bufs[slot].astype(jnp.float32)

            @pl.when(op == REPLACE)
            def _():
                out_ref[...] = bufs[slot].astype(jnp.float32)

            if s + 2 <= N_STEPS:
                cr = pltpu.make_async_remote_copy(
                    src_ref=tsrc, dst_ref=tdst.at[s + 1],
                    send_sem=tsend.at[s + 1], recv_sem=trecv.at[s + 1],
                    device_id=(ptn_ref[s + 1],),
                    device_id_type=pl.DeviceIdType.MESH,
                )
                cr.start()
                cr.wait_send()

        y = out_ref[...]
        out_ref[...] = y * jax.nn.sigmoid(y)

    return pl.pallas_call(
        body,
        out_shape=jax.ShapeDtypeStruct((m, n), jnp.float32),
        in_specs=[
            pl.BlockSpec(memory_space=pltpu.SMEM),
            pl.BlockSpec(memory_space=pltpu.SMEM),
            pl.BlockSpec(memory_space=pltpu.VMEM),
            pl.BlockSpec(memory_space=pltpu.VMEM),
        ],
        out_specs=pl.BlockSpec(memory_space=pltpu.VMEM),
        scratch_shapes=[
            pltpu.VMEM((m, n), jnp.bfloat16),
            pltpu.VMEM((2, m, n), jnp.bfloat16),
            pltpu.VMEM((8, 128), jnp.float32),
            pltpu.VMEM((N_STEPS, 8, 128), jnp.float32),
            pltpu.SemaphoreType.DMA((N_STEPS,)),
            pltpu.SemaphoreType.DMA((N_STEPS,)),
            pltpu.SemaphoreType.DMA((N_STEPS,)),
            pltpu.SemaphoreType.DMA((N_STEPS,)),
        ],
        compiler_params=pltpu.CompilerParams(
            collective_id=0, vmem_limit_bytes=100 * 1024 * 1024
        ),
    )(ptn, opc, x, w_mat)
